# baseline (speedup 1.0000x reference)
import jax
import jax.numpy as jnp
from jax import lax
from jax.experimental import pallas as pl
from jax.experimental.pallas import tpu as pltpu
from jax.experimental.pallas import tpu_sc as plsc

B = 4096
N = 64
HID = 128
K = 16
BK = B * K

GA = 128


GA2 = GA // 2


def _topk_idx_body(h_ref, idx_ref):
    h = h_ref[...]
    keys = jnp.max(h, axis=2)
    kv = jnp.concatenate([keys[:GA2], keys[GA2:]], axis=1)
    lane = lax.broadcasted_iota(jnp.int32, (1, HID), 1)
    llane = lane & (N - 1)
    nid = jnp.broadcast_to(llane, (GA2, HID))
    size = 2
    while size <= N:
        stride = size // 2
        while stride >= 1:
            upper = (llane & stride) != 0
            keep_small = jnp.logical_xor((llane & stride) == 0,
                                         (llane & size) != 0)
            kp = jnp.where(upper, pltpu.roll(kv, stride, 1),
                           pltpu.roll(kv, HID - stride, 1))
            np_ = jnp.where(upper, pltpu.roll(nid, stride, 1),
                            pltpu.roll(nid, HID - stride, 1))
            self_small = (kv < kp) | ((kv == kp) & (nid > np_))
            keep_self = self_small == keep_small
            kv = jnp.where(keep_self, kv, kp)
            nid = jnp.where(keep_self, nid, np_)
            stride //= 2
        size *= 2
    g0 = pl.program_id(0) * GA
    graph = g0 + lax.broadcasted_iota(jnp.int32, (GA2, K), 0)
    idx_ref[:GA2, :] = (graph * N) + nid[:, N - K:N]
    idx_ref[GA2:, :] = ((g0 + GA2) * N) + (
        lax.broadcasted_iota(jnp.int32, (GA2, K), 0) * N + nid[:, HID - K:])


def _topk_idx(h):
    return pl.pallas_call(
        _topk_idx_body,
        grid=(B // GA,),
        in_specs=[pl.BlockSpec((GA, N, HID), lambda i: (i, 0, 0))],
        out_specs=pl.BlockSpec((GA, K), lambda i: (i, 0)),
        out_shape=jax.ShapeDtypeStruct((B, K), jnp.int32),
    )(h)


NC, NS = 2, 16
NW = NC * NS
ROWS_PER_W = BK // NW
CH = 128
NCH = ROWS_PER_W // CH


def _sc_gather_body(tbl_ref, idx_ref, out_ref, idx_v, rows_v, sem):
    wid = lax.axis_index("s") * NC + lax.axis_index("c")
    base = wid * ROWS_PER_W
    for i in range(NCH):
        off = base + i * CH
        pltpu.sync_copy(idx_ref.at[pl.ds(off, CH)], idx_v)
        pltpu.async_copy(tbl_ref.at[idx_v], rows_v, sem).wait()
        pltpu.sync_copy(rows_v, out_ref.at[pl.ds(off, CH)])


import functools


@functools.lru_cache(maxsize=1)
def _sc_gather():
    return pl.kernel(
        _sc_gather_body,
        out_type=jax.ShapeDtypeStruct((BK, HID), jnp.float32),
        mesh=plsc.VectorSubcoreMesh(core_axis_name="c", subcore_axis_name="s"),
        scratch_types=[
            pltpu.VMEM((CH,), jnp.int32),
            pltpu.VMEM((CH, HID), jnp.float32),
            pltpu.SemaphoreType.DMA,
        ],
    )


GC = 64
R = GC * K


GCH = 8
RCH = GCH * K

_BITREV = [int(f"{i:07b}"[::-1], 2) for i in range(HID)]


def _sublane_bitonic_sort(xt):
    p = lax.broadcasted_iota(jnp.int32, (HID, 1), 0)
    size = 2
    while size <= HID:
        stride = size // 2
        while stride >= 1:
            dphys = 64 // stride
            sbit = 64 // size
            upper = (p & dphys) != 0
            keep_small = jnp.logical_xor((p & dphys) == 0, (p & sbit) != 0)
            prt = jnp.where(upper, pltpu.roll(xt, dphys, 0),
                            pltpu.roll(xt, HID - dphys, 0))
            xt = jnp.where(keep_small, jnp.minimum(xt, prt),
                           jnp.maximum(xt, prt))
            stride //= 2
        size *= 2
    return xt


def _sort_attn_body(x_ref, q_ref, w1p_ref, w2_ref, p2_ref, out_ref, acc_ref):
    w1p = w1p_ref[...].reshape(1, 1, HID)
    w2 = w2_ref[...]

    def chunk(c, _):
        x = x_ref[pl.ds(c * RCH, RCH), :]
        xs = _sublane_bitonic_sort(x.T).T
        x3 = xs.reshape(GCH, K, HID)
        dot = jnp.sum(x3 * w1p, axis=2)
        q = q_ref[pl.ds(c * GCH, GCH), :]
        qdot = jnp.sum(q * w2, axis=1, keepdims=True)
        logit = dot + qdot
        logit = jnp.where(logit >= 0, logit, 0.01 * logit)
        mx = jnp.max(logit, axis=1, keepdims=True)
        e = jnp.exp(logit - mx)
        wgt = e / jnp.sum(e, axis=1, keepdims=True)
        acc_ref[pl.ds(c * GCH, GCH), :] = jnp.sum(x3 * wgt[:, :, None], axis=1)
        return 0

    lax.fori_loop(0, GC // GCH, chunk, 0, unroll=2)
    acc = acc_ref[...]
    hi = acc.astype(jnp.bfloat16)
    lo = (acc - hi.astype(jnp.float32)).astype(jnp.bfloat16)
    p2 = p2_ref[...]
    dn = (((1,), (0,)), ((), ()))
    out_ref[...] = (
        lax.dot_general(hi, p2, dn, preferred_element_type=jnp.float32)
        + lax.dot_general(lo, p2, dn, preferred_element_type=jnp.float32))


def _sort_attn(pooled, attention_query, w1p, w2, p2):
    return pl.pallas_call(
        _sort_attn_body,
        grid=(B // GC,),
        in_specs=[
            pl.BlockSpec((R, HID), lambda i: (i, 0)),
            pl.BlockSpec((GC, HID), lambda i: (i, 0)),
            pl.BlockSpec((1, HID), lambda i: (0, 0)),
            pl.BlockSpec((1, HID), lambda i: (0, 0)),
            pl.BlockSpec((HID, HID), lambda i: (0, 0)),
        ],
        out_specs=pl.BlockSpec((GC, HID), lambda i: (i, 0)),
        out_shape=jax.ShapeDtypeStruct((B, HID), jnp.float32),
        scratch_shapes=[pltpu.VMEM((GC, HID), jnp.float32)],
    )(pooled, attention_query, w1p, w2, p2)


def kernel(h, attention_query, W_att):
    idx = _topk_idx(h)
    pooled = _sc_gather()(h.reshape(B * N, HID), idx.reshape(BK))
    perm = jnp.asarray(_BITREV, dtype=jnp.int32)
    w1p = W_att[:HID, 0][perm].reshape(1, HID)
    w2 = W_att[HID:, 0].reshape(1, HID)
    p2 = jnp.zeros((HID, HID), jnp.bfloat16).at[perm, jnp.arange(HID)].set(1.0)
    return _sort_attn(pooled, attention_query, w1p, w2, p2)


# speedup vs baseline: 1.0472x; 1.0472x over previous
import jax
import jax.numpy as jnp
from jax import lax
from jax.experimental import pallas as pl
from jax.experimental.pallas import tpu as pltpu
from jax.experimental.pallas import tpu_sc as plsc

B = 4096
N = 64
HID = 128
K = 16
BK = B * K

GA = 256


GA2 = GA // 2


def _topk_idx_body(h_ref, idx_ref):
    h = h_ref[...]
    keys = jnp.max(h, axis=2)
    kv = jnp.concatenate([keys[:GA2], keys[GA2:]], axis=1)
    lane = lax.broadcasted_iota(jnp.int32, (1, HID), 1)
    llane = lane & (N - 1)
    nid = jnp.broadcast_to(llane, (GA2, HID))
    size = 2
    while size <= N:
        stride = size // 2
        while stride >= 1:
            upper = (llane & stride) != 0
            keep_small = jnp.logical_xor((llane & stride) == 0,
                                         (llane & size) != 0)
            kp = jnp.where(upper, pltpu.roll(kv, stride, 1),
                           pltpu.roll(kv, HID - stride, 1))
            np_ = jnp.where(upper, pltpu.roll(nid, stride, 1),
                            pltpu.roll(nid, HID - stride, 1))
            self_small = (kv < kp) | ((kv == kp) & (nid > np_))
            keep_self = self_small == keep_small
            kv = jnp.where(keep_self, kv, kp)
            nid = jnp.where(keep_self, nid, np_)
            stride //= 2
        size *= 2
    g0 = pl.program_id(0) * GA
    graph = g0 + lax.broadcasted_iota(jnp.int32, (GA2, K), 0)
    idx_ref[:GA2, :] = (graph * N) + nid[:, N - K:N]
    idx_ref[GA2:, :] = ((g0 + GA2) * N) + (
        lax.broadcasted_iota(jnp.int32, (GA2, K), 0) * N + nid[:, HID - K:])


def _topk_idx(h):
    return pl.pallas_call(
        _topk_idx_body,
        grid=(B // GA,),
        in_specs=[pl.BlockSpec((GA, N, HID), lambda i: (i, 0, 0))],
        out_specs=pl.BlockSpec((GA, K), lambda i: (i, 0)),
        out_shape=jax.ShapeDtypeStruct((B, K), jnp.int32),
    )(h)


NC, NS = 2, 16
NW = NC * NS
ROWS_PER_W = BK // NW
CH = 128
NCH = ROWS_PER_W // CH


def _sc_gather_body(tbl_ref, idx_ref, out_ref, idx_v, rows_v, sem):
    wid = lax.axis_index("s") * NC + lax.axis_index("c")
    base = wid * ROWS_PER_W
    for i in range(NCH):
        off = base + i * CH
        pltpu.sync_copy(idx_ref.at[pl.ds(off, CH)], idx_v)
        pltpu.async_copy(tbl_ref.at[idx_v], rows_v, sem).wait()
        pltpu.sync_copy(rows_v, out_ref.at[pl.ds(off, CH)])


import functools


@functools.lru_cache(maxsize=1)
def _sc_gather():
    return pl.kernel(
        _sc_gather_body,
        out_type=jax.ShapeDtypeStruct((BK, HID), jnp.float32),
        mesh=plsc.VectorSubcoreMesh(core_axis_name="c", subcore_axis_name="s"),
        scratch_types=[
            pltpu.VMEM((CH,), jnp.int32),
            pltpu.VMEM((CH, HID), jnp.float32),
            pltpu.SemaphoreType.DMA,
        ],
    )


GC = 64
R = GC * K


GCH = 8
RCH = GCH * K

_BITREV = [int(f"{i:07b}"[::-1], 2) for i in range(HID)]


def _sublane_bitonic_sort(xt):
    p = lax.broadcasted_iota(jnp.int32, (HID, 1), 0)
    size = 2
    while size <= HID:
        stride = size // 2
        while stride >= 1:
            dphys = 64 // stride
            sbit = 64 // size
            upper = (p & dphys) != 0
            keep_small = jnp.logical_xor((p & dphys) == 0, (p & sbit) != 0)
            prt = jnp.where(upper, pltpu.roll(xt, dphys, 0),
                            pltpu.roll(xt, HID - dphys, 0))
            xt = jnp.where(keep_small, jnp.minimum(xt, prt),
                           jnp.maximum(xt, prt))
            stride //= 2
        size *= 2
    return xt


def _sort_attn_body(x_ref, q_ref, w1p_ref, w2_ref, p2_ref, out_ref, acc_ref):
    w1p = w1p_ref[...].reshape(1, 1, HID)
    w2 = w2_ref[...]

    def chunk(c, _):
        x = x_ref[pl.ds(c * RCH, RCH), :]
        xs = _sublane_bitonic_sort(x.T).T
        x3 = xs.reshape(GCH, K, HID)
        dot = jnp.sum(x3 * w1p, axis=2)
        q = q_ref[pl.ds(c * GCH, GCH), :]
        qdot = jnp.sum(q * w2, axis=1, keepdims=True)
        logit = dot + qdot
        logit = jnp.where(logit >= 0, logit, 0.01 * logit)
        mx = jnp.max(logit, axis=1, keepdims=True)
        e = jnp.exp(logit - mx)
        wgt = e / jnp.sum(e, axis=1, keepdims=True)
        acc_ref[pl.ds(c * GCH, GCH), :] = jnp.sum(x3 * wgt[:, :, None], axis=1)
        return 0

    lax.fori_loop(0, GC // GCH, chunk, 0, unroll=2)
    acc = acc_ref[...]
    hi = acc.astype(jnp.bfloat16)
    lo = (acc - hi.astype(jnp.float32)).astype(jnp.bfloat16)
    p2 = p2_ref[...]
    dn = (((1,), (0,)), ((), ()))
    out_ref[...] = (
        lax.dot_general(hi, p2, dn, preferred_element_type=jnp.float32)
        + lax.dot_general(lo, p2, dn, preferred_element_type=jnp.float32))


def _sort_attn(pooled, attention_query, w1p, w2, p2):
    return pl.pallas_call(
        _sort_attn_body,
        grid=(B // GC,),
        in_specs=[
            pl.BlockSpec((R, HID), lambda i: (i, 0)),
            pl.BlockSpec((GC, HID), lambda i: (i, 0)),
            pl.BlockSpec((1, HID), lambda i: (0, 0)),
            pl.BlockSpec((1, HID), lambda i: (0, 0)),
            pl.BlockSpec((HID, HID), lambda i: (0, 0)),
        ],
        out_specs=pl.BlockSpec((GC, HID), lambda i: (i, 0)),
        out_shape=jax.ShapeDtypeStruct((B, HID), jnp.float32),
        scratch_shapes=[pltpu.VMEM((GC, HID), jnp.float32)],
    )(pooled, attention_query, w1p, w2, p2)


def kernel(h, attention_query, W_att):
    idx = _topk_idx(h)
    pooled = _sc_gather()(h.reshape(B * N, HID), idx.reshape(BK))
    perm = jnp.asarray(_BITREV, dtype=jnp.int32)
    w1p = W_att[:HID, 0][perm].reshape(1, HID)
    w2 = W_att[HID:, 0].reshape(1, HID)
    p2 = jnp.zeros((HID, HID), jnp.bfloat16).at[perm, jnp.arange(HID)].set(1.0)
    return _sort_attn(pooled, attention_query, w1p, w2, p2)
